# baseline (speedup 1.0000x reference)
import dataclasses
import functools

import numpy as np
import jax
import jax.numpy as jnp
from jax.experimental import pallas as pl
from jax.experimental.pallas import tpu as pltpu
from jax.experimental.pallas import tpu_sc as plsc

TOKENS = 8192
HIDDEN = 2048
EXPERTS = 64

R_SC = 2048
BM = 1024
SC_CH = 8
SC_NW = 32


def _sc_convert(x):
    mesh = plsc.VectorSubcoreMesh(core_axis_name="c", subcore_axis_name="s")
    cp = pltpu.CompilerParams()
    if "needs_layout_passes" in pltpu.CompilerParams.__dataclass_fields__:
        cp = dataclasses.replace(cp, needs_layout_passes=False)

    rows_w = R_SC // SC_NW
    nch = rows_w // SC_CH
    chw = SC_CH * HIDDEN

    @functools.partial(
        pl.kernel,
        out_type=jax.ShapeDtypeStruct((R_SC * HIDDEN // 2,), jnp.int32),
        mesh=mesh,
        compiler_params=cp,
        scratch_types=[
            pltpu.VMEM((chw,), jnp.float32),
            pltpu.VMEM((chw,), jnp.float32),
            pltpu.VMEM((chw // 2,), jnp.int32),
            pltpu.VMEM((chw // 2,), jnp.int32),
            pltpu.SemaphoreType.DMA,
            pltpu.SemaphoreType.DMA,
            pltpu.SemaphoreType.DMA,
            pltpu.SemaphoreType.DMA,
        ],
    )
    def conv(x_hbm, o_hbm, in0, in1, ou0, ou1, si0, si1, so0, so1):
        wid = jax.lax.axis_index("s") * 2 + jax.lax.axis_index("c")
        base = wid * (rows_w * HIDDEN)

        def convert(iv, ov):
            for u in range(chw // 32):
                a = iv[pl.ds(u * 32, 16)]
                b = iv[pl.ds(u * 32 + 16, 16)]
                ov[pl.ds(u * 16, 16)] = plsc.bitcast(
                    plsc.pack(a, b, format=plsc.PackFormat.INTERLEAVED),
                    jnp.int32)

        def in_cp(g, buf, sem):
            off = pl.multiple_of(base + g * chw, 8)
            return pltpu.make_async_copy(x_hbm.at[pl.ds(off, chw)], buf, sem)

        def out_cp(g, buf, sem):
            off = pl.multiple_of((base + g * chw) // 2, 8)
            return pltpu.make_async_copy(
                buf, o_hbm.at[pl.ds(off, chw // 2)], sem)

        in_cp(0, in0, si0).start()
        in_cp(1, in1, si1).start()

        @pl.loop(0, nch, step=2)
        def _(g):
            in_cp(g, in0, si0).wait()
            convert(in0, ou0)

            @pl.when(g >= 2)
            def _():
                out_cp(g - 2, ou0, so0).wait()
            out_cp(g, ou0, so0).start()

            @pl.when(g + 2 < nch)
            def _():
                in_cp(g + 2, in0, si0).start()

            in_cp(g + 1, in1, si1).wait()
            convert(in1, ou1)

            @pl.when(g >= 2)
            def _():
                out_cp(g - 1, ou1, so1).wait()
            out_cp(g + 1, ou1, so1).start()

            @pl.when(g + 3 < nch)
            def _():
                in_cp(g + 3, in1, si1).start()

        out_cp(nch - 2, ou0, so0).wait()
        out_cp(nch - 1, ou1, so1).wait()

    packed = conv(x.reshape(-1))
    return jax.lax.bitcast_convert_type(
        packed.reshape(R_SC, HIDDEN // 2), jnp.bfloat16).reshape(R_SC, HIDDEN)


def _mm_body(x_ref, w_ref, o_ref):
    x = x_ref[...]
    if x.dtype != jnp.bfloat16:
        x = x.astype(jnp.bfloat16)
    w = w_ref[...].astype(jnp.bfloat16)
    o_ref[...] = jax.lax.dot_general(
        x, w, (((1,), (1,)), ((), ())),
        preferred_element_type=jnp.float32)


def _tc_matmul(x, w, row_start, rows):
    grid = (rows // BM,)
    off = row_start // BM
    return pl.pallas_call(
        _mm_body,
        grid=grid,
        in_specs=[
            pl.BlockSpec((BM, HIDDEN), lambda i: (i + off, 0)),
            pl.BlockSpec((EXPERTS, HIDDEN), lambda i: (0, 0)),
        ],
        out_specs=pl.BlockSpec((BM, EXPERTS), lambda i: (i, 0)),
        out_shape=jax.ShapeDtypeStruct((rows, EXPERTS), jnp.float32),
    )(x, w)


@functools.partial(jax.jit, static_argnames=())
def kernel(x, W):
    xb = _sc_convert(x)
    out_hi = _tc_matmul(x, W, R_SC, TOKENS - R_SC)
    w_perm = (W.reshape(EXPERTS, HIDDEN // 32, 2, 16)
              .transpose(0, 1, 3, 2).reshape(EXPERTS, HIDDEN))
    out_lo = _tc_matmul(xb, w_perm, 0, R_SC)
    return jnp.concatenate([out_lo, out_hi], axis=0)


# speedup vs baseline: 6.4864x; 6.4864x over previous
import functools

import jax
import jax.numpy as jnp
from jax.experimental import pallas as pl


def _gate_body(x_ref, w_ref, o_ref):
    x = x_ref[...].astype(jnp.bfloat16)
    w = w_ref[...].astype(jnp.bfloat16)
    o_ref[...] = jax.lax.dot_general(
        x, w, (((1,), (1,)), ((), ())),
        preferred_element_type=jnp.float32)


@functools.partial(jax.jit, static_argnames=())
def kernel(x, W):
    tokens, hidden = x.shape
    experts = W.shape[0]
    bm = 512
    return pl.pallas_call(
        _gate_body,
        grid=(tokens // bm,),
        in_specs=[
            pl.BlockSpec((bm, hidden), lambda i: (i, 0)),
            pl.BlockSpec((experts, hidden), lambda i: (0, 0)),
        ],
        out_specs=pl.BlockSpec((bm, experts), lambda i: (i, 0)),
        out_shape=jax.ShapeDtypeStruct((tokens, experts), jnp.float32),
    )(x, W)
